# baseline (speedup 1.0000x reference)
import jax
import jax.numpy as jnp
from jax import lax
from jax.experimental import pallas as pl
from jax.experimental.pallas import tpu as pltpu
from jax.experimental.pallas import tpu_sc as plsc

D = 64
SCALE = 8.0
B_TOKENS = 16384
SEQ = 50
B = B_TOKENS * SEQ
NC = 2
NS = 16
NW = NC * NS
B_PER_W = B // NW
R = 512
N_CHUNK = B_PER_W // R


def _emb_body(idx_hbm, lut_hbm, out_hbm, idx_v, rows_v, sem):
    wid = lax.axis_index("s") * NC + lax.axis_index("c")
    base = wid * B_PER_W

    def chunk(i, carry):
        off = pl.multiple_of(base + i * R, R)
        pltpu.sync_copy(idx_hbm.at[pl.ds(off, R)], idx_v)
        pltpu.async_copy(lut_hbm.at[idx_v], rows_v, sem).wait()

        def scale_rows(r, c2):
            for u in range(4):
                for c in range(D // 16):
                    sl = (4 * r + u, pl.ds(c * 16, 16))
                    rows_v[sl] = rows_v[sl] * SCALE
            return c2

        lax.fori_loop(0, R // 4, scale_rows, 0)
        pltpu.sync_copy(rows_v, out_hbm.at[pl.ds(off, R)])
        return carry

    lax.fori_loop(0, N_CHUNK, chunk, 0)


def kernel(x, lut):
    idx = jnp.reshape(x, (B,)).astype(jnp.int32)
    mesh = plsc.VectorSubcoreMesh(core_axis_name="c", subcore_axis_name="s")
    out = pl.kernel(
        _emb_body,
        mesh=mesh,
        out_type=jax.ShapeDtypeStruct((B, D), jnp.float32),
        scratch_types=[
            pltpu.VMEM((R,), jnp.int32),
            pltpu.VMEM((R, D), jnp.float32),
            pltpu.SemaphoreType.DMA,
        ],
        compiler_params=pltpu.CompilerParams(use_tc_tiling_on_sc=False),
    )(idx, lut)
    return jnp.reshape(out, (B_TOKENS, SEQ, D))


# speedup vs baseline: 1.0907x; 1.0907x over previous
import jax
import jax.numpy as jnp
from jax import lax
from jax.experimental import pallas as pl
from jax.experimental.pallas import tpu as pltpu
from jax.experimental.pallas import tpu_sc as plsc

D = 64
SCALE = 8.0
B_TOKENS = 16384
SEQ = 50
B = B_TOKENS * SEQ
NC = 2
NS = 16
NW = NC * NS
B_PER_W = B // NW
R = 800
N_CHUNK = B_PER_W // R
N_PAIR = N_CHUNK // 2


def _scale_chunk(rows):
    @plsc.parallel_loop(0, R, 1, unroll=8)
    def _(r):
        for c in range(D // 16):
            sl = (r, pl.ds(c * 16, 16))
            rows[sl] = rows[sl] * SCALE


def _emb_body(idx_hbm, lut_hbm, out_hbm, idx_v, rows0, rows1, gsem0, gsem1,
              osem0, osem1):
    wid = lax.axis_index("s") * NC + lax.axis_index("c")
    base = wid * B_PER_W

    pltpu.sync_copy(idx_hbm.at[pl.ds(base, B_PER_W)], idx_v)

    def gather(chunk, rows, gsem):
        idx_sl = idx_v.at[pl.ds(chunk * R, R)]
        pltpu.make_async_copy(lut_hbm.at[idx_sl], rows, gsem).start()

    def out_desc(chunk, rows, osem):
        off = pl.multiple_of(base + chunk * R, R)
        return pltpu.make_async_copy(rows, out_hbm.at[pl.ds(off, R)], osem)

    gather(0, rows0, gsem0)

    def pair(j, carry):
        a = 2 * j

        pltpu.make_async_copy(lut_hbm.at[idx_v.at[pl.ds(a * R, R)]], rows0,
                              gsem0).wait()

        @pl.when(j >= 1)
        def _():
            out_desc(a - 1, rows1, osem1).wait()

        gather(a + 1, rows1, gsem1)
        _scale_chunk(rows0)
        out_desc(a, rows0, osem0).start()

        pltpu.make_async_copy(lut_hbm.at[idx_v.at[pl.ds((a + 1) * R, R)]],
                              rows1, gsem1).wait()

        @pl.when(j < N_PAIR - 1)
        def _():
            out_desc(a, rows0, osem0).wait()
            gather(a + 2, rows0, gsem0)

        _scale_chunk(rows1)
        out_desc(a + 1, rows1, osem1).start()
        return carry

    lax.fori_loop(0, N_PAIR, pair, 0)

    out_desc(N_CHUNK - 2, rows0, osem0).wait()
    out_desc(N_CHUNK - 1, rows1, osem1).wait()


def kernel(x, lut):
    idx = jnp.reshape(x, (B,)).astype(jnp.int32)
    mesh = plsc.VectorSubcoreMesh(core_axis_name="c", subcore_axis_name="s")
    out = pl.kernel(
        _emb_body,
        mesh=mesh,
        out_type=jax.ShapeDtypeStruct((B, D), jnp.float32),
        scratch_types=[
            pltpu.VMEM((B_PER_W,), jnp.int32),
            pltpu.VMEM((R, D), jnp.float32),
            pltpu.VMEM((R, D), jnp.float32),
            pltpu.SemaphoreType.DMA,
            pltpu.SemaphoreType.DMA,
            pltpu.SemaphoreType.DMA,
            pltpu.SemaphoreType.DMA,
        ],
        compiler_params=pltpu.CompilerParams(use_tc_tiling_on_sc=False),
    )(idx, lut)
    return jnp.reshape(out, (B_TOKENS, SEQ, D))
